# baseline (speedup 1.0000x reference)
import functools

import jax
import jax.numpy as jnp
from jax import lax
from jax.experimental import pallas as pl
from jax.experimental.pallas import tpu as pltpu
from jax.experimental.pallas import tpu_sc as plsc

_B = 64 * 1024
_D = 768
_V = 1024
_C = 64


@functools.cache
def _build_gather():
    info = plsc.get_sparse_core_info()
    num_cores, num_subcores = info.num_cores, info.num_subcores
    num_workers = num_cores * num_subcores
    b_per_w = _B // num_workers
    n_chunks = b_per_w // _C
    mesh = plsc.VectorSubcoreMesh(core_axis_name="c", subcore_axis_name="s")

    @functools.partial(
        pl.kernel,
        mesh=mesh,
        out_type=jax.ShapeDtypeStruct((_B, _D), jnp.float32),
        scratch_types=[
            pltpu.VMEM((b_per_w,), jnp.int32),
            pltpu.VMEM((_C, _D), jnp.float32),
            pltpu.VMEM((_C, _D), jnp.float32),
            pltpu.SemaphoreType.DMA,
            pltpu.SemaphoreType.DMA,
            pltpu.SemaphoreType.DMA,
            pltpu.SemaphoreType.DMA,
        ],
    )
    def gather_kernel(idx_hbm, table_hbm, out_hbm, idx_v, rows0, rows1,
                      gsem0, gsem1, wsem0, wsem1):
        sid = lax.axis_index("s")
        wid = sid * num_cores + lax.axis_index("c")
        base = wid * b_per_w
        pltpu.sync_copy(idx_hbm.at[pl.ds(base, b_per_w)], idx_v)

        rows = (rows0, rows1)
        gsem = (gsem0, gsem1)
        wsem = (wsem0, wsem1)

        def start_gather(g, b):
            pltpu.async_copy(
                table_hbm.at[idx_v.at[pl.ds(g * _C, _C)]], rows[b], gsem[b])

        def wait_gather(b):
            pltpu.make_async_copy(
                table_hbm.at[pl.ds(0, _C)], rows[b], gsem[b]).wait()

        def start_write(g, b):
            pltpu.async_copy(
                rows[b], out_hbm.at[pl.ds(base + g * _C, _C)], wsem[b])

        def wait_write(b):
            pltpu.make_async_copy(
                rows[b], out_hbm.at[pl.ds(base, _C)], wsem[b]).wait()

        start_gather(0, 0)
        start_gather(1, 1)
        wait_gather(0)
        start_write(0, 0)

        def body(i, carry):
            wait_write(0)
            start_gather(2 * i + 2, 0)
            wait_gather(1)
            start_write(2 * i + 1, 1)
            wait_write(1)
            start_gather(2 * i + 3, 1)
            wait_gather(0)
            start_write(2 * i + 2, 0)
            return carry

        lax.fori_loop(0, n_chunks // 2 - 1, body, 0)

        wait_gather(1)
        start_write(n_chunks - 1, 1)
        wait_write(0)
        wait_write(1)

    return gather_kernel


def kernel(position_ids, pos_embed):
    idx = position_ids.reshape(-1)
    out = _build_gather()(idx, pos_embed)
    return out.reshape(position_ids.shape + (pos_embed.shape[1],))
